# baseline (speedup 1.0000x reference)
import functools

import jax
import jax.numpy as jnp
from jax import lax
from jax.experimental import pallas as pl
from jax.experimental.pallas import tpu as pltpu
from jax.experimental.pallas import tpu_sc as plsc

B, C, H, W, M = 64, 8, 128, 128, 128
HW = H * W
CHW = C * HW
N = B * M
NT = 16
EPW = N // NT
VPW = EPW // 16
NCH = EPW // 128


@functools.partial(
    pl.kernel,
    mesh=plsc.VectorSubcoreMesh(core_axis_name="c", subcore_axis_name="s"),
    out_type=[jax.ShapeDtypeStruct((16,), jnp.float32),
              jax.ShapeDtypeStruct((NT, 2, 16), jnp.float32)],
    scratch_types=[
        pltpu.VMEM((EPW,), jnp.int32),
        pltpu.VMEM((EPW,), jnp.int32),
        pltpu.VMEM((EPW,), jnp.float32),
        pltpu.VMEM((EPW,), jnp.float32),
        pltpu.VMEM((NCH, 128), jnp.int32),
        pltpu.VMEM((NCH, 128), jnp.float32),
        pltpu.VMEM((2, 16), jnp.float32),
        pltpu.VMEM((NT, 2, 16), jnp.float32),
        pltpu.VMEM((16,), jnp.float32),
        pltpu.SemaphoreType.DMA,
        pltpu.SemaphoreType.DMA,
        pltpu.SemaphoreType.DMA,
    ],
)
def _depth_loss(feat_hbm, ind_hbm, cat_hbm, tgt_hbm, msk_hbm,
                out_hbm, parts_hbm,
                ind_v, cat_v, tgt_v, msk_v, gidx_v, vals_v,
                part_v, allp_v, stage, sem_i, sem_f, sem_g):
    cid = lax.axis_index("c")
    sid = lax.axis_index("s")

    @pl.when(cid == 0)
    def _():
        base = sid * EPW
        cp_ind = pltpu.async_copy(ind_hbm.at[pl.ds(base, EPW)], ind_v, sem_i)
        cp_cat = pltpu.async_copy(cat_hbm.at[pl.ds(base, EPW)], cat_v, sem_i)
        cp_tgt = pltpu.async_copy(tgt_hbm.at[pl.ds(base, EPW)], tgt_v, sem_f)
        cp_msk = pltpu.async_copy(msk_hbm.at[pl.ds(base, EPW)], msk_v, sem_f)
        cp_ind.wait()
        cp_cat.wait()
        b0 = sid * (EPW // M)
        for v in range(VPW):
            g = (ind_v[pl.ds(v * 16, 16)]
                 + cat_v[pl.ds(v * 16, 16)] * HW
                 + (b0 + v // 8) * CHW)
            gidx_v[v // 8, pl.ds((v % 8) * 16, 16)] = g
        gathers = [
            pltpu.async_copy(feat_hbm.at[gidx_v.at[j]], vals_v.at[j], sem_g)
            for j in range(NCH)
        ]
        cp_tgt.wait()
        cp_msk.wait()
        for cp in gathers:
            cp.wait()
        acc = jnp.zeros((16,), jnp.float32)
        mac = jnp.zeros((16,), jnp.float32)
        for v in range(VPW):
            val = vals_v[v // 8, pl.ds((v % 8) * 16, 16)]
            m = msk_v[pl.ds(v * 16, 16)]
            t = tgt_v[pl.ds(v * 16, 16)]
            acc = acc + jnp.abs(val * m - t * m)
            mac = mac + m
        part_v[0, :] = acc
        part_v[1, :] = mac
        pltpu.sync_copy(part_v, parts_hbm.at[sid])
        plsc.subcore_barrier()

        @pl.when(sid == 0)
        def _():
            pltpu.sync_copy(parts_hbm, allp_v)
            facc = jnp.zeros((16,), jnp.float32)
            fmac = jnp.zeros((16,), jnp.float32)
            for i in range(NT):
                facc = facc + allp_v[i, 0, :]
                fmac = fmac + allp_v[i, 1, :]
            lanes = lax.iota(jnp.int32, 16)
            dnums = lax.GatherDimensionNumbers(
                offset_dims=(), collapsed_slice_dims=(0,),
                start_index_map=(0,))
            shuffle = lambda x, perm: lax.gather(
                x, perm[:, None], dnums, slice_sizes=(1,),
                mode=lax.GatherScatterMode.PROMISE_IN_BOUNDS)
            for k in (1, 2, 4, 8):
                perm = lanes ^ k
                facc = facc + shuffle(facc, perm)
                fmac = fmac + shuffle(fmac, perm)
            res = facc / (fmac + 1e-6)
            stage[...] = res
            pltpu.sync_copy(stage, out_hbm)


def kernel(output, target, ind, mask, cat):
    feat = output.reshape(-1)
    ind32 = ind.astype(jnp.int32).reshape(-1)
    cat32 = cat.astype(jnp.int32).reshape(-1)
    tgt = target.reshape(-1)
    msk = mask.reshape(-1)
    res, _ = _depth_loss(feat, ind32, cat32, tgt, msk)
    return res[0]


# speedup vs baseline: 1.0549x; 1.0549x over previous
import functools

import jax
import jax.numpy as jnp
from jax import lax
from jax.experimental import pallas as pl
from jax.experimental.pallas import tpu as pltpu
from jax.experimental.pallas import tpu_sc as plsc

B, C, H, W, M = 64, 8, 128, 128, 128
HW = H * W
CHW = C * HW
N = B * M
NT = 16
EPW = N // NT
VPW = EPW // 16
NCH = EPW // 128


@functools.partial(
    pl.kernel,
    mesh=plsc.VectorSubcoreMesh(core_axis_name="c", subcore_axis_name="s",
                                num_cores=1),
    out_type=[jax.ShapeDtypeStruct((16,), jnp.float32),
              jax.ShapeDtypeStruct((NT, 2, 16), jnp.float32)],
    scratch_types=[
        pltpu.VMEM((EPW,), jnp.int32),
        pltpu.VMEM((EPW,), jnp.int32),
        pltpu.VMEM((EPW,), jnp.float32),
        pltpu.VMEM((EPW,), jnp.float32),
        pltpu.VMEM((NCH, 128), jnp.int32),
        pltpu.VMEM((NCH, 128), jnp.float32),
        pltpu.VMEM((2, 16), jnp.float32),
        pltpu.VMEM((NT, 2, 16), jnp.float32),
        pltpu.VMEM((16,), jnp.float32),
        pltpu.SemaphoreType.DMA,
        pltpu.SemaphoreType.DMA,
        pltpu.SemaphoreType.DMA,
    ],
)
def _depth_loss(feat_hbm, ind_hbm, cat_hbm, tgt_hbm, msk_hbm,
                out_hbm, parts_hbm,
                ind_v, cat_v, tgt_v, msk_v, gidx_v, vals_v,
                part_v, allp_v, stage, sem_i, sem_f, sem_g):
    cid = lax.axis_index("c")
    sid = lax.axis_index("s")

    @pl.when(cid == 0)
    def _():
        base = sid * EPW
        cp_ind = pltpu.async_copy(ind_hbm.at[pl.ds(base, EPW)], ind_v, sem_i)
        cp_cat = pltpu.async_copy(cat_hbm.at[pl.ds(base, EPW)], cat_v, sem_i)
        cp_tgt = pltpu.async_copy(tgt_hbm.at[pl.ds(base, EPW)], tgt_v, sem_f)
        cp_msk = pltpu.async_copy(msk_hbm.at[pl.ds(base, EPW)], msk_v, sem_f)
        cp_ind.wait()
        cp_cat.wait()
        b0 = sid * (EPW // M)
        for v in range(VPW):
            g = (ind_v[pl.ds(v * 16, 16)]
                 + cat_v[pl.ds(v * 16, 16)] * HW
                 + (b0 + v // 8) * CHW)
            gidx_v[v // 8, pl.ds((v % 8) * 16, 16)] = g
        gathers = [
            pltpu.async_copy(feat_hbm.at[gidx_v.at[j]], vals_v.at[j], sem_g)
            for j in range(NCH)
        ]
        cp_tgt.wait()
        cp_msk.wait()
        for cp in gathers:
            cp.wait()
        acc = jnp.zeros((16,), jnp.float32)
        mac = jnp.zeros((16,), jnp.float32)
        for v in range(VPW):
            val = vals_v[v // 8, pl.ds((v % 8) * 16, 16)]
            m = msk_v[pl.ds(v * 16, 16)]
            t = tgt_v[pl.ds(v * 16, 16)]
            acc = acc + jnp.abs(val * m - t * m)
            mac = mac + m
        part_v[0, :] = acc
        part_v[1, :] = mac
        pltpu.sync_copy(part_v, parts_hbm.at[sid])
        plsc.subcore_barrier()

        @pl.when(sid == 0)
        def _():
            pltpu.sync_copy(parts_hbm, allp_v)
            facc = jnp.zeros((16,), jnp.float32)
            fmac = jnp.zeros((16,), jnp.float32)
            for i in range(NT):
                facc = facc + allp_v[i, 0, :]
                fmac = fmac + allp_v[i, 1, :]
            lanes = lax.iota(jnp.int32, 16)
            dnums = lax.GatherDimensionNumbers(
                offset_dims=(), collapsed_slice_dims=(0,),
                start_index_map=(0,))
            shuffle = lambda x, perm: lax.gather(
                x, perm[:, None], dnums, slice_sizes=(1,),
                mode=lax.GatherScatterMode.PROMISE_IN_BOUNDS)
            for k in (1, 2, 4, 8):
                perm = lanes ^ k
                facc = facc + shuffle(facc, perm)
                fmac = fmac + shuffle(fmac, perm)
            res = facc / (fmac + 1e-6)
            stage[...] = res
            pltpu.sync_copy(stage, out_hbm)


def kernel(output, target, ind, mask, cat):
    feat = output.reshape(-1)
    ind32 = ind.astype(jnp.int32).reshape(-1)
    cat32 = cat.astype(jnp.int32).reshape(-1)
    tgt = target.reshape(-1)
    msk = mask.reshape(-1)
    res, _ = _depth_loss(feat, ind32, cat32, tgt, msk)
    return res[0]
